# Initial kernel scaffold; baseline (speedup 1.0000x reference)
import jax
import jax.numpy as jnp
from jax.experimental import pallas as pl


def kernel(x, edge_index, neighbors, W1, W2):
    raise NotImplementedError("write your pallas kernel here")


# speedup vs baseline: 4.7258x; 4.7258x over previous
import functools

import jax
import jax.numpy as jnp
from jax import lax
from jax.experimental import pallas as pl
from jax.experimental.pallas import tpu as pltpu
from jax.experimental.pallas import tpu_sc as plsc

_N = 10000
_E = 320000
_D = 128
_H = 256
_HH = _H // 2
_OUT = 512
_NB = 25
_NBP = 32
_NPAD = 10240
_NT = 16
_NC = 2
_CH = 80
_EPT = _E // _NT
_NF = _N * _NBP
_CPT = _EPT // _CH
_GE = 4000
_GC = _GE // _CH
_RPT = _NPAD // _NT
_ZR = 32


def _mm1(x, w1):
  def body(x_ref, w_ref, o_ref):
    o_ref[...] = jnp.dot(x_ref[...], w_ref[...],
                         preferred_element_type=jnp.float32)
  return pl.pallas_call(
      body,
      grid=(25,),
      in_specs=[
          pl.BlockSpec((400, _D), lambda i: (i, 0)),
          pl.BlockSpec((_D, _H), lambda i: (0, 0)),
      ],
      out_specs=pl.BlockSpec((400, _H), lambda i: (i, 0)),
      out_shape=jax.ShapeDtypeStruct((_N, _H), jnp.float32),
  )(x, w1)


def _mm2(agg, nbsum, deg, w2):
  def body(a0_ref, a1_ref, n0_ref, n1_ref, d_ref, w_ref, o_ref):
    r = 1.0 / jnp.maximum(d_ref[...], 1.0)
    h0 = jnp.maximum(a0_ref[...] * r + n0_ref[...] * (1.0 / _NB), 0.0)
    h1 = jnp.maximum(a1_ref[...] * r + n1_ref[...] * (1.0 / _NB), 0.0)
    w = w_ref[...]
    o_ref[...] = (
        jnp.dot(h0, w[:_HH], preferred_element_type=jnp.float32)
        + jnp.dot(h1, w[_HH:], preferred_element_type=jnp.float32))
  nblk = _NPAD // 512
  return pl.pallas_call(
      body,
      grid=(nblk,),
      in_specs=[
          pl.BlockSpec((512, _HH), lambda i: (i, 0)),
          pl.BlockSpec((512, _HH), lambda i: (i + nblk, 0)),
          pl.BlockSpec((512, _HH), lambda i: (i, 0)),
          pl.BlockSpec((512, _HH), lambda i: (i + nblk, 0)),
          pl.BlockSpec((512, 1), lambda i: (i, 0)),
          pl.BlockSpec((_H, _OUT), lambda i: (0, 0)),
      ],
      out_specs=pl.BlockSpec((512, _OUT), lambda i: (i, 0)),
      out_shape=jax.ShapeDtypeStruct((_NPAD, _OUT), jnp.float32),
  )(agg, agg, nbsum, nbsum, deg, w2)


def _sc_aggregate(h2n, src2, dst, nbs2, nbdst):
  mesh = plsc.VectorSubcoreMesh(core_axis_name="c", subcore_axis_name="s")
  out_type = (
      jax.ShapeDtypeStruct((_NC * _NPAD, _HH), jnp.float32),
      jax.ShapeDtypeStruct((_NC * _NPAD, _HH), jnp.float32),
      jax.ShapeDtypeStruct((_NPAD,), jnp.float32),
  )
  scratch = [
      pltpu.VMEM_SHARED((_NPAD, _HH), jnp.float32),
      pltpu.VMEM_SHARED((_NPAD,), jnp.float32),
      pltpu.VMEM((_GE,), jnp.int32),
      pltpu.VMEM((_GE,), jnp.int32),
      pltpu.VMEM((_CH, _HH), jnp.float32),
      pltpu.VMEM((_CH, _HH), jnp.float32),
      pltpu.VMEM((_ZR, _HH), jnp.float32),
      pltpu.VMEM((_RPT,), jnp.float32),
      pltpu.VMEM((_CH,), jnp.float32),
      pltpu.SemaphoreType.DMA,
      pltpu.SemaphoreType.DMA,
  ]

  @functools.partial(pl.kernel, out_type=out_type, mesh=mesh,
                     scratch_types=scratch)
  def body(h2n_r, src2_r, dst_r, nbs2_r, nbdst_r, agg_o, nb_o, deg_o,
           acc_sh, deg_sh, gbuf_v, sbuf_v,
           rows0_v, rows1_v, zs_v, dstage_v, ones_v, sem0, sem1):
    rows = (rows0_v, rows1_v)
    sems = (sem0, sem1)
    c = lax.axis_index("c")
    s = lax.axis_index("s")
    rbase = s * _RPT
    zvec = jnp.zeros((16,), jnp.float32)

    def fill_zs(i, carry):
      for j in range(_HH // 16):
        zs_v[i, pl.ds(j * 16, 16)] = zvec
      return carry

    lax.fori_loop(0, _ZR, fill_zs, 0)
    def zdrow(i, carry):
      dstage_v[pl.ds(i * 16, 16)] = zvec
      return carry
    lax.fori_loop(0, _RPT // 16, zdrow, 0)
    ovec = jnp.ones((16,), jnp.float32)
    def orow(i, carry):
      ones_v[pl.ds(i * 16, 16)] = ovec
      return carry
    lax.fori_loop(0, _CH // 16, orow, 0)

    for k in range(_RPT // _ZR):
      pltpu.sync_copy(zs_v, acc_sh.at[pl.ds(rbase + k * _ZR, _ZR)])
    @pl.when(c == 0)
    def _():
      pltpu.sync_copy(dstage_v, deg_sh.at[pl.ds(rbase, _RPT)])
    plsc.subcore_barrier()

    def run_phase(idx_r, idx_base, dst_r_, dst_base, do_deg):
      def gslice(j):
        return gbuf_v.at[pl.ds(pl.multiple_of(j * _CH, 8), _CH)]
      def sslice(j):
        return sbuf_v.at[pl.ds(pl.multiple_of(j * _CH, 8), _CH)]
      def start(slot, j):
        pltpu.async_copy(h2n_r.at[gslice(j)], rows[slot], sems[slot])
      def finish(slot, j):
        pltpu.make_async_copy(h2n_r.at[gslice(j)], rows[slot],
                              sems[slot]).wait()
        pltpu.sync_copy(rows[slot], acc_sh.at[sslice(j)], add=True)
        if do_deg:
          @pl.when(c == 0)
          def _():
            pltpu.sync_copy(ones_v, deg_sh.at[sslice(j)], add=True)
      def group(g, carry):
        o1 = pl.multiple_of(idx_base + g * _GE, 8)
        o2 = pl.multiple_of(dst_base + g * _GE, 8)
        pltpu.sync_copy(idx_r.at[pl.ds(o1, _GE)], gbuf_v)
        pltpu.sync_copy(dst_r_.at[pl.ds(o2, _GE)], sbuf_v)
        start(0, 0)
        start(1, 1)
        def pair(k, carry2):
          j0 = 2 * k
          finish(0, j0)
          @pl.when(j0 + 2 < _GC)
          def _():
            start(0, j0 + 2)
          finish(1, j0 + 1)
          @pl.when(j0 + 3 < _GC)
          def _():
            start(1, j0 + 3)
          return carry2
        lax.fori_loop(0, _GC // 2, pair, 0)
        return carry
      lax.fori_loop(0, _CPT // _GC, group, 0)

    run_phase(src2_r, c * _E + s * _EPT, dst_r, s * _EPT, True)
    plsc.subcore_barrier()

    obase = c * _NPAD + rbase
    for k in range(_RPT // _ZR):
      pltpu.sync_copy(acc_sh.at[pl.ds(rbase + k * _ZR, _ZR)], zs_v)
      pltpu.sync_copy(zs_v, agg_o.at[pl.ds(obase + k * _ZR, _ZR)])
    @pl.when(c == 0)
    def _():
      sl = pl.ds(rbase, _RPT)
      pltpu.sync_copy(deg_sh.at[sl], dstage_v)
      pltpu.sync_copy(dstage_v, deg_o.at[sl])
    lax.fori_loop(0, _ZR, fill_zs, 0)
    for k in range(_RPT // _ZR):
      pltpu.sync_copy(zs_v, acc_sh.at[pl.ds(rbase + k * _ZR, _ZR)])
    plsc.subcore_barrier()

    run_phase(nbs2_r, c * _NF + s * _EPT, nbdst_r, s * _EPT, False)
    plsc.subcore_barrier()

    for k in range(_RPT // _ZR):
      pltpu.sync_copy(acc_sh.at[pl.ds(rbase + k * _ZR, _ZR)], zs_v)
      pltpu.sync_copy(zs_v, nb_o.at[pl.ds(obase + k * _ZR, _ZR)])

  return body(h2n, src2, dst, nbs2, nbdst)


def kernel(x, edge_index, neighbors, W1, W2):
  src = edge_index[0]
  dst = edge_index[1]
  h = _mm1(x, W1)
  h2n = h.reshape(_N * 2, _HH)

  src2 = jnp.concatenate([2 * src, 2 * src + 1])
  pad = jnp.broadcast_to(jnp.arange(_N, dtype=jnp.int32)[:, None],
                         (_N, _NBP - _NB))
  nbf = jnp.concatenate([neighbors, pad], axis=1).reshape(-1)
  nbs2 = jnp.concatenate([2 * nbf, 2 * nbf + 1])
  p = jnp.arange(_NF, dtype=jnp.int32)
  nbdst = jnp.where((p & (_NBP - 1)) < _NB, p >> 5, _N + (p % (_NPAD - _N)))

  agg, nbsum, deg = _sc_aggregate(h2n, src2, dst, nbs2, nbdst)
  z = _mm2(agg, nbsum, deg.reshape(_NPAD, 1), W2)
  return z[:_N]


# speedup vs baseline: 5.5672x; 1.1780x over previous
import functools

import jax
import jax.numpy as jnp
from jax import lax
from jax.experimental import pallas as pl
from jax.experimental.pallas import tpu as pltpu
from jax.experimental.pallas import tpu_sc as plsc

_N = 10000
_E = 320000
_D = 128
_H = 256
_HH = _H // 2
_OUT = 512
_NB = 25
_NBP = 32
_NPAD = 10240
_NT = 16
_NC = 2
_CH = 80
_CPT = 255
_EPT = _CPT * _CH
_EPC = _EPT * _NT
_NF = _N * _NBP
_GC = 51
_GE = _GC * _CH
_NS = 3
_RPT = _NPAD // _NT
_ZR = 32


def _mm1(x, w1):
  def body(x_ref, w_ref, o_ref):
    o_ref[...] = jnp.dot(x_ref[...], w_ref[...],
                         preferred_element_type=jnp.float32)
  return pl.pallas_call(
      body,
      grid=(25,),
      in_specs=[
          pl.BlockSpec((400, _D), lambda i: (i, 0)),
          pl.BlockSpec((_D, _H), lambda i: (0, 0)),
      ],
      out_specs=pl.BlockSpec((400, _H), lambda i: (i, 0)),
      out_shape=jax.ShapeDtypeStruct((_N, _H), jnp.float32),
  )(x, w1)


def _mm2(agg, nbsum, deg, w2):
  def body(a0_ref, a1_ref, n0_ref, n1_ref, d_ref, w_ref, o_ref):
    r = 1.0 / jnp.maximum(d_ref[...], 1.0)
    h0 = jnp.maximum(a0_ref[...] * r + n0_ref[...] * (1.0 / _NB), 0.0)
    h1 = jnp.maximum(a1_ref[...] * r + n1_ref[...] * (1.0 / _NB), 0.0)
    w = w_ref[...]
    o_ref[...] = (
        jnp.dot(h0, w[:_HH], preferred_element_type=jnp.float32)
        + jnp.dot(h1, w[_HH:], preferred_element_type=jnp.float32))
  nblk = _NPAD // 512
  return pl.pallas_call(
      body,
      grid=(nblk,),
      in_specs=[
          pl.BlockSpec((512, _HH), lambda i: (i, 0)),
          pl.BlockSpec((512, _HH), lambda i: (i + nblk, 0)),
          pl.BlockSpec((512, _HH), lambda i: (i, 0)),
          pl.BlockSpec((512, _HH), lambda i: (i + nblk, 0)),
          pl.BlockSpec((512, 1), lambda i: (i, 0)),
          pl.BlockSpec((_H, _OUT), lambda i: (0, 0)),
      ],
      out_specs=pl.BlockSpec((512, _OUT), lambda i: (i, 0)),
      out_shape=jax.ShapeDtypeStruct((_NPAD, _OUT), jnp.float32),
  )(agg, agg, nbsum, nbsum, deg, w2)


def _sc_aggregate(h2n, src2, dst, nbs2, nbdst):
  mesh = plsc.VectorSubcoreMesh(core_axis_name="c", subcore_axis_name="s")
  out_type = (
      jax.ShapeDtypeStruct((_NC * _NPAD, _HH), jnp.float32),
      jax.ShapeDtypeStruct((_NC * _NPAD, _HH), jnp.float32),
      jax.ShapeDtypeStruct((_NPAD,), jnp.float32),
  )
  scratch = [
      pltpu.VMEM_SHARED((_NPAD, _HH), jnp.float32),
      pltpu.VMEM_SHARED((_NPAD,), jnp.float32),
      pltpu.VMEM((_GE,), jnp.int32),
      pltpu.VMEM((_GE,), jnp.int32),
      pltpu.VMEM((_CH, _HH), jnp.float32),
      pltpu.VMEM((_CH, _HH), jnp.float32),
      pltpu.VMEM((_CH, _HH), jnp.float32),
      pltpu.VMEM((_ZR, _HH), jnp.float32),
      pltpu.VMEM((_RPT,), jnp.float32),
      pltpu.VMEM((_CH,), jnp.float32),
      pltpu.SemaphoreType.DMA,
      pltpu.SemaphoreType.DMA,
      pltpu.SemaphoreType.DMA,
      pltpu.SemaphoreType.DMA,
      pltpu.SemaphoreType.DMA,
      pltpu.SemaphoreType.DMA,
      pltpu.SemaphoreType.DMA,
  ]

  @functools.partial(pl.kernel, out_type=out_type, mesh=mesh,
                     scratch_types=scratch)
  def body(h2n_r, src2_r, dst_r, nbs2_r, nbdst_r, agg_o, nb_o, deg_o,
           acc_sh, deg_sh, gbuf_v, sbuf_v,
           rows0_v, rows1_v, rows2_v, zs_v, dstage_v, ones_v,
           g0, g1, g2, s0, s1, s2, dsem):
    rows = (rows0_v, rows1_v, rows2_v)
    gsem = (g0, g1, g2)
    ssem = (s0, s1, s2)
    c = lax.axis_index("c")
    s = lax.axis_index("s")
    rbase = s * _RPT
    zvec = jnp.zeros((16,), jnp.float32)

    def fill_zs(i, carry):
      for j in range(_HH // 16):
        zs_v[i, pl.ds(j * 16, 16)] = zvec
      return carry

    lax.fori_loop(0, _ZR, fill_zs, 0)
    def zdrow(i, carry):
      dstage_v[pl.ds(i * 16, 16)] = zvec
      return carry
    lax.fori_loop(0, _RPT // 16, zdrow, 0)
    ovec = jnp.ones((16,), jnp.float32)
    def orow(i, carry):
      ones_v[pl.ds(i * 16, 16)] = ovec
      return carry
    lax.fori_loop(0, _CH // 16, orow, 0)

    for k in range(_RPT // _ZR):
      pltpu.sync_copy(zs_v, acc_sh.at[pl.ds(rbase + k * _ZR, _ZR)])
    @pl.when(c == 0)
    def _():
      pltpu.sync_copy(dstage_v, deg_sh.at[pl.ds(rbase, _RPT)])
    plsc.subcore_barrier()

    def run_phase(idx_r, idx_base, dst_r_, dst_base, do_deg):
      def gslice(j):
        return gbuf_v.at[pl.ds(pl.multiple_of(j * _CH, 8), _CH)]
      def sslice(j):
        return sbuf_v.at[pl.ds(pl.multiple_of(j * _CH, 8), _CH)]
      def start(slot, j):
        pltpu.async_copy(h2n_r.at[gslice(j)], rows[slot], gsem[slot])
      def wait_scatter(slot, j):
        pltpu.make_async_copy(rows[slot], acc_sh.at[sslice(j)],
                              ssem[slot]).wait()
      def wait_deg(j):
        pltpu.make_async_copy(ones_v, deg_sh.at[sslice(j)], dsem).wait()
      def step(slot, slot2, j):
        t = j + 2
        @pl.when(t < _GC)
        def _():
          @pl.when(t >= _NS)
          def _():
            wait_scatter(slot2, j - 1)
          start(slot2, t)
        pltpu.make_async_copy(h2n_r.at[gslice(j)], rows[slot],
                              gsem[slot]).wait()
        pltpu.async_copy(rows[slot], acc_sh.at[sslice(j)], ssem[slot],
                         add=True)
        if do_deg:
          @pl.when(c == 0)
          def _():
            pltpu.async_copy(ones_v, deg_sh.at[sslice(j)], dsem,
                             add=True)
      def group(g, carry):
        o1 = pl.multiple_of(idx_base + g * _GE, 8)
        o2 = pl.multiple_of(dst_base + g * _GE, 8)
        pltpu.sync_copy(idx_r.at[pl.ds(o1, _GE)], gbuf_v)
        pltpu.sync_copy(dst_r_.at[pl.ds(o2, _GE)], sbuf_v)
        start(0, 0)
        start(1, 1)
        def triple(k, carry2):
          j0 = 3 * k
          step(0, 2, j0)
          step(1, 0, j0 + 1)
          step(2, 1, j0 + 2)
          return carry2
        lax.fori_loop(0, _GC // _NS, triple, 0)
        for slot, j in ((0, _GC - 3), (1, _GC - 2), (2, _GC - 1)):
          wait_scatter(slot, j)
        if do_deg:
          @pl.when(c == 0)
          def _():
            def dw(i, carry3):
              wait_deg(0)
              return carry3
            lax.fori_loop(0, _GC, dw, 0)
        return carry
      lax.fori_loop(0, _CPT // _GC, group, 0)

    run_phase(src2_r, c * _EPC + s * _EPT, dst_r, s * _EPT, True)
    plsc.subcore_barrier()

    obase = c * _NPAD + rbase
    for k in range(_RPT // _ZR):
      pltpu.sync_copy(acc_sh.at[pl.ds(rbase + k * _ZR, _ZR)], zs_v)
      pltpu.sync_copy(zs_v, agg_o.at[pl.ds(obase + k * _ZR, _ZR)])
    @pl.when(c == 0)
    def _():
      sl = pl.ds(rbase, _RPT)
      pltpu.sync_copy(deg_sh.at[sl], dstage_v)
      pltpu.sync_copy(dstage_v, deg_o.at[sl])
    lax.fori_loop(0, _ZR, fill_zs, 0)
    for k in range(_RPT // _ZR):
      pltpu.sync_copy(zs_v, acc_sh.at[pl.ds(rbase + k * _ZR, _ZR)])
    plsc.subcore_barrier()

    run_phase(nbs2_r, c * _EPC + s * _EPT, nbdst_r, s * _EPT, False)
    plsc.subcore_barrier()

    for k in range(_RPT // _ZR):
      pltpu.sync_copy(acc_sh.at[pl.ds(rbase + k * _ZR, _ZR)], zs_v)
      pltpu.sync_copy(zs_v, nb_o.at[pl.ds(obase + k * _ZR, _ZR)])

  return body(h2n, src2, dst, nbs2, nbdst)


def kernel(x, edge_index, neighbors, W1, W2):
  src = edge_index[0]
  dst = edge_index[1]
  h = _mm1(x, W1)
  h2n = h.reshape(_N * 2, _HH)

  npe = _EPC - _E
  padg = jnp.arange(npe, dtype=jnp.int32) % _N
  pads = _N + (jnp.arange(npe, dtype=jnp.int32) % (_NPAD - _N))
  src2 = jnp.concatenate([2 * src, 2 * padg, 2 * src + 1, 2 * padg + 1])
  dst2 = jnp.concatenate([dst, pads])
  pad = jnp.broadcast_to(jnp.arange(_N, dtype=jnp.int32)[:, None],
                         (_N, _NBP - _NB))
  nbf = jnp.concatenate([neighbors, pad], axis=1).reshape(-1)
  nbs2 = jnp.concatenate([2 * nbf, 2 * padg, 2 * nbf + 1, 2 * padg + 1])
  p = jnp.arange(_NF, dtype=jnp.int32)
  nbd = jnp.where((p & (_NBP - 1)) < _NB, p >> 5, _N + (p % (_NPAD - _N)))
  nbdst = jnp.concatenate([nbd, pads])

  agg, nbsum, deg = _sc_aggregate(h2n, src2, dst2, nbs2, nbdst)
  z = _mm2(agg, nbsum, deg.reshape(_NPAD, 1), W2)
  return z[:_N]


# speedup vs baseline: 5.7965x; 1.0412x over previous
import functools

import jax
import jax.numpy as jnp
from jax import lax
from jax.experimental import pallas as pl
from jax.experimental.pallas import tpu as pltpu
from jax.experimental.pallas import tpu_sc as plsc

_N = 10000
_E = 320000
_D = 128
_H = 256
_HH = _H // 2
_OUT = 512
_NB = 25
_NBP = 32
_NPAD = 10240
_NT = 16
_NC = 2
_CH = 80
_CPT = 255
_EPT = _CPT * _CH
_EPC = _EPT * _NT
_CPT2 = 204
_FPT = _CPT2 * _CH
_FPC = _FPT * _NT
_NF = _N * _NB
_GC = 51
_GE = _GC * _CH
_NS = 3
_RPT = _NPAD // _NT
_ZR = 32


def _mm1(x, w1):
  def body(x_ref, w_ref, o_ref):
    o_ref[...] = jnp.dot(x_ref[...], w_ref[...],
                         preferred_element_type=jnp.float32)
  return pl.pallas_call(
      body,
      grid=(25,),
      in_specs=[
          pl.BlockSpec((400, _D), lambda i: (i, 0)),
          pl.BlockSpec((_D, _H), lambda i: (0, 0)),
      ],
      out_specs=pl.BlockSpec((400, _H), lambda i: (i, 0)),
      out_shape=jax.ShapeDtypeStruct((_N, _H), jnp.float32),
  )(x, w1)


def _mm2(agg, nbsum, deg, w2):
  def body(a0_ref, a1_ref, n0_ref, n1_ref, d_ref, w_ref, o_ref):
    r = 1.0 / jnp.maximum(d_ref[...], 1.0)
    h0 = jnp.maximum(a0_ref[...] * r + n0_ref[...] * (1.0 / _NB), 0.0)
    h1 = jnp.maximum(a1_ref[...] * r + n1_ref[...] * (1.0 / _NB), 0.0)
    w = w_ref[...]
    o_ref[...] = (
        jnp.dot(h0, w[:_HH], preferred_element_type=jnp.float32)
        + jnp.dot(h1, w[_HH:], preferred_element_type=jnp.float32))
  nblk = _NPAD // 512
  return pl.pallas_call(
      body,
      grid=(nblk,),
      in_specs=[
          pl.BlockSpec((512, _HH), lambda i: (i, 0)),
          pl.BlockSpec((512, _HH), lambda i: (i + nblk, 0)),
          pl.BlockSpec((512, _HH), lambda i: (i, 0)),
          pl.BlockSpec((512, _HH), lambda i: (i + nblk, 0)),
          pl.BlockSpec((512, 1), lambda i: (i, 0)),
          pl.BlockSpec((_H, _OUT), lambda i: (0, 0)),
      ],
      out_specs=pl.BlockSpec((512, _OUT), lambda i: (i, 0)),
      out_shape=jax.ShapeDtypeStruct((_NPAD, _OUT), jnp.float32),
  )(agg, agg, nbsum, nbsum, deg, w2)


def _sc_aggregate(h2n, src2, dst, nbs2, nbdst):
  mesh = plsc.VectorSubcoreMesh(core_axis_name="c", subcore_axis_name="s")
  out_type = (
      jax.ShapeDtypeStruct((_NC * _NPAD, _HH), jnp.float32),
      jax.ShapeDtypeStruct((_NC * _NPAD, _HH), jnp.float32),
      jax.ShapeDtypeStruct((_NPAD,), jnp.float32),
  )
  scratch = [
      pltpu.VMEM_SHARED((_NPAD, _HH), jnp.float32),
      pltpu.VMEM_SHARED((_NPAD,), jnp.float32),
      pltpu.VMEM((_GE,), jnp.int32),
      pltpu.VMEM((_GE,), jnp.int32),
      pltpu.VMEM((_CH, _HH), jnp.float32),
      pltpu.VMEM((_CH, _HH), jnp.float32),
      pltpu.VMEM((_CH, _HH), jnp.float32),
      pltpu.VMEM((_ZR, _HH), jnp.float32),
      pltpu.VMEM((_RPT,), jnp.float32),
      pltpu.VMEM((_CH,), jnp.float32),
      pltpu.SemaphoreType.DMA,
      pltpu.SemaphoreType.DMA,
      pltpu.SemaphoreType.DMA,
      pltpu.SemaphoreType.DMA,
      pltpu.SemaphoreType.DMA,
      pltpu.SemaphoreType.DMA,
      pltpu.SemaphoreType.DMA,
  ]

  @functools.partial(pl.kernel, out_type=out_type, mesh=mesh,
                     scratch_types=scratch)
  def body(h2n_r, src2_r, dst_r, nbs2_r, nbdst_r, agg_o, nb_o, deg_o,
           acc_sh, deg_sh, gbuf_v, sbuf_v,
           rows0_v, rows1_v, rows2_v, zs_v, dstage_v, ones_v,
           g0, g1, g2, s0, s1, s2, dsem):
    rows = (rows0_v, rows1_v, rows2_v)
    gsem = (g0, g1, g2)
    ssem = (s0, s1, s2)
    c = lax.axis_index("c")
    s = lax.axis_index("s")
    rbase = s * _RPT
    zvec = jnp.zeros((16,), jnp.float32)

    def fill_zs(i, carry):
      for j in range(_HH // 16):
        zs_v[i, pl.ds(j * 16, 16)] = zvec
      return carry

    lax.fori_loop(0, _ZR, fill_zs, 0)
    def zdrow(i, carry):
      dstage_v[pl.ds(i * 16, 16)] = zvec
      return carry
    lax.fori_loop(0, _RPT // 16, zdrow, 0)
    ovec = jnp.ones((16,), jnp.float32)
    def orow(i, carry):
      ones_v[pl.ds(i * 16, 16)] = ovec
      return carry
    lax.fori_loop(0, _CH // 16, orow, 0)

    for k in range(_RPT // _ZR):
      pltpu.sync_copy(zs_v, acc_sh.at[pl.ds(rbase + k * _ZR, _ZR)])
    @pl.when(c == 0)
    def _():
      pltpu.sync_copy(dstage_v, deg_sh.at[pl.ds(rbase, _RPT)])
    plsc.subcore_barrier()

    def run_phase(idx_r, idx_base, dst_r_, dst_base, ngroups, do_deg):
      def gslice(j):
        return gbuf_v.at[pl.ds(pl.multiple_of(j * _CH, 8), _CH)]
      def sslice(j):
        return sbuf_v.at[pl.ds(pl.multiple_of(j * _CH, 8), _CH)]
      def start(slot, j):
        pltpu.async_copy(h2n_r.at[gslice(j)], rows[slot], gsem[slot])
      def wait_scatter(slot, j):
        pltpu.make_async_copy(rows[slot], acc_sh.at[sslice(j)],
                              ssem[slot]).wait()
      def wait_deg(j):
        pltpu.make_async_copy(ones_v, deg_sh.at[sslice(j)], dsem).wait()
      def step(slot, slot2, j):
        t = j + 2
        @pl.when(t < _GC)
        def _():
          @pl.when(t >= _NS)
          def _():
            wait_scatter(slot2, j - 1)
          start(slot2, t)
        pltpu.make_async_copy(h2n_r.at[gslice(j)], rows[slot],
                              gsem[slot]).wait()
        pltpu.async_copy(rows[slot], acc_sh.at[sslice(j)], ssem[slot],
                         add=True)
        if do_deg:
          @pl.when(c == 0)
          def _():
            pltpu.async_copy(ones_v, deg_sh.at[sslice(j)], dsem,
                             add=True)
      def group(g, carry):
        o1 = pl.multiple_of(idx_base + g * _GE, 8)
        o2 = pl.multiple_of(dst_base + g * _GE, 8)
        pltpu.sync_copy(idx_r.at[pl.ds(o1, _GE)], gbuf_v)
        pltpu.sync_copy(dst_r_.at[pl.ds(o2, _GE)], sbuf_v)
        start(0, 0)
        start(1, 1)
        def triple(k, carry2):
          j0 = 3 * k
          step(0, 2, j0)
          step(1, 0, j0 + 1)
          step(2, 1, j0 + 2)
          return carry2
        lax.fori_loop(0, _GC // _NS, triple, 0)
        for slot, j in ((0, _GC - 3), (1, _GC - 2), (2, _GC - 1)):
          wait_scatter(slot, j)
        if do_deg:
          @pl.when(c == 0)
          def _():
            def dw(i, carry3):
              wait_deg(0)
              return carry3
            lax.fori_loop(0, _GC, dw, 0)
        return carry
      lax.fori_loop(0, ngroups, group, 0)

    run_phase(src2_r, c * _EPC + s * _EPT, dst_r, s * _EPT,
              _CPT // _GC, True)
    plsc.subcore_barrier()

    obase = c * _NPAD + rbase
    for k in range(_RPT // _ZR):
      pltpu.sync_copy(acc_sh.at[pl.ds(rbase + k * _ZR, _ZR)], zs_v)
      pltpu.sync_copy(zs_v, agg_o.at[pl.ds(obase + k * _ZR, _ZR)])
    @pl.when(c == 0)
    def _():
      sl = pl.ds(rbase, _RPT)
      pltpu.sync_copy(deg_sh.at[sl], dstage_v)
      pltpu.sync_copy(dstage_v, deg_o.at[sl])
    lax.fori_loop(0, _ZR, fill_zs, 0)
    for k in range(_RPT // _ZR):
      pltpu.sync_copy(zs_v, acc_sh.at[pl.ds(rbase + k * _ZR, _ZR)])
    plsc.subcore_barrier()

    run_phase(nbs2_r, c * _FPC + s * _FPT, nbdst_r, s * _FPT,
              _CPT2 // _GC, False)
    plsc.subcore_barrier()

    for k in range(_RPT // _ZR):
      pltpu.sync_copy(acc_sh.at[pl.ds(rbase + k * _ZR, _ZR)], zs_v)
      pltpu.sync_copy(zs_v, nb_o.at[pl.ds(obase + k * _ZR, _ZR)])

  return body(h2n, src2, dst, nbs2, nbdst)


def kernel(x, edge_index, neighbors, W1, W2):
  src = edge_index[0]
  dst = edge_index[1]
  h = _mm1(x, W1)
  h2n = h.reshape(_N * 2, _HH)

  npe = _EPC - _E
  padg = jnp.arange(npe, dtype=jnp.int32) % _N
  pads = _N + (jnp.arange(npe, dtype=jnp.int32) % (_NPAD - _N))
  src2 = jnp.concatenate([2 * src, 2 * padg, 2 * src + 1, 2 * padg + 1])
  dst2 = jnp.concatenate([dst, pads])
  npf = _FPC - _NF
  padgf = jnp.arange(npf, dtype=jnp.int32) % _N
  padsf = _N + (jnp.arange(npf, dtype=jnp.int32) % (_NPAD - _N))
  nbf = neighbors.reshape(-1)
  nbs2 = jnp.concatenate(
      [2 * nbf, 2 * padgf, 2 * nbf + 1, 2 * padgf + 1])
  nbd = jnp.repeat(jnp.arange(_N, dtype=jnp.int32), _NB)
  nbdst = jnp.concatenate([nbd, padsf])

  agg, nbsum, deg = _sc_aggregate(h2n, src2, dst2, nbs2, nbdst)
  z = _mm2(agg, nbsum, deg.reshape(_NPAD, 1), W2)
  return z[:_N]


# speedup vs baseline: 9.9565x; 1.7177x over previous
import functools

import jax
import jax.numpy as jnp
from jax import lax
from jax.experimental import pallas as pl
from jax.experimental.pallas import tpu as pltpu
from jax.experimental.pallas import tpu_sc as plsc

_N = 10000
_E = 320000
_D = 128
_H = 256
_OUT = 512
_NB = 25
_NPAD = 10240
_NT = 16
_NC = 2
_CH = 80
_CPT = 126
_EPT = _CPT * _CH
_EPC = _EPT * _NT
_GC1 = 42
_CPT2 = 99
_FPT = _CPT2 * _CH
_FPC = _FPT * _NT
_GC2 = 33
_NF = _N * _NB
_GE = _GC1 * _CH
_NS = 3
_RPT = _NPAD // _NT
_ZR = 32


def _mm2(agg, nbsum, deg, w1, w2):
  def body(a0_ref, a1_ref, n0_ref, n1_ref, d0_ref, d1_ref,
           w1_ref, w2_ref, o_ref):
    r = 1.0 / jnp.maximum(d0_ref[...] + d1_ref[...], 1.0)
    u = (a0_ref[...] + a1_ref[...]) * r \
        + (n0_ref[...] + n1_ref[...]) * (1.0 / _NB)
    h2 = jnp.maximum(
        jnp.dot(u, w1_ref[...], preferred_element_type=jnp.float32), 0.0)
    o_ref[...] = jnp.dot(h2, w2_ref[...], preferred_element_type=jnp.float32)
  nblk = _NPAD // 512
  return pl.pallas_call(
      body,
      grid=(nblk,),
      in_specs=[
          pl.BlockSpec((512, _D), lambda i: (i, 0)),
          pl.BlockSpec((512, _D), lambda i: (i + nblk, 0)),
          pl.BlockSpec((512, _D), lambda i: (i, 0)),
          pl.BlockSpec((512, _D), lambda i: (i + nblk, 0)),
          pl.BlockSpec((512, 1), lambda i: (i, 0)),
          pl.BlockSpec((512, 1), lambda i: (i + nblk, 0)),
          pl.BlockSpec((_D, _H), lambda i: (0, 0)),
          pl.BlockSpec((_H, _OUT), lambda i: (0, 0)),
      ],
      out_specs=pl.BlockSpec((512, _OUT), lambda i: (i, 0)),
      out_shape=jax.ShapeDtypeStruct((_NPAD, _OUT), jnp.float32),
  )(agg, agg, nbsum, nbsum, deg, deg, w1, w2)


def _sc_aggregate(x, srcp, dstp, nbp, nbdst):
  mesh = plsc.VectorSubcoreMesh(core_axis_name="c", subcore_axis_name="s")
  out_type = (
      jax.ShapeDtypeStruct((_NC * _NPAD, _D), jnp.float32),
      jax.ShapeDtypeStruct((_NC * _NPAD, _D), jnp.float32),
      jax.ShapeDtypeStruct((_NC * _NPAD,), jnp.float32),
  )
  scratch = [
      pltpu.VMEM_SHARED((_NPAD, _D), jnp.float32),
      pltpu.VMEM_SHARED((_NPAD,), jnp.float32),
      pltpu.VMEM((_GE,), jnp.int32),
      pltpu.VMEM((_GE,), jnp.int32),
      pltpu.VMEM((_CH, _D), jnp.float32),
      pltpu.VMEM((_CH, _D), jnp.float32),
      pltpu.VMEM((_CH, _D), jnp.float32),
      pltpu.VMEM((_ZR, _D), jnp.float32),
      pltpu.VMEM((_RPT,), jnp.float32),
      pltpu.VMEM((_CH,), jnp.float32),
      pltpu.SemaphoreType.DMA,
      pltpu.SemaphoreType.DMA,
      pltpu.SemaphoreType.DMA,
      pltpu.SemaphoreType.DMA,
      pltpu.SemaphoreType.DMA,
      pltpu.SemaphoreType.DMA,
      pltpu.SemaphoreType.DMA,
  ]

  @functools.partial(pl.kernel, out_type=out_type, mesh=mesh,
                     scratch_types=scratch)
  def body(x_r, src_r, dst_r, nbp_r, nbdst_r, agg_o, nb_o, deg_o,
           acc_sh, deg_sh, gbuf_v, sbuf_v,
           rows0_v, rows1_v, rows2_v, zs_v, dstage_v, ones_v,
           g0, g1, g2, s0, s1, s2, dsem):
    rows = (rows0_v, rows1_v, rows2_v)
    gsem = (g0, g1, g2)
    ssem = (s0, s1, s2)
    c = lax.axis_index("c")
    s = lax.axis_index("s")
    rbase = s * _RPT
    zvec = jnp.zeros((16,), jnp.float32)

    def fill_zs(i, carry):
      for j in range(_D // 16):
        zs_v[i, pl.ds(j * 16, 16)] = zvec
      return carry

    lax.fori_loop(0, _ZR, fill_zs, 0)
    def zdrow(i, carry):
      dstage_v[pl.ds(i * 16, 16)] = zvec
      return carry
    lax.fori_loop(0, _RPT // 16, zdrow, 0)
    ovec = jnp.ones((16,), jnp.float32)
    def orow(i, carry):
      ones_v[pl.ds(i * 16, 16)] = ovec
      return carry
    lax.fori_loop(0, _CH // 16, orow, 0)

    for k in range(_RPT // _ZR):
      pltpu.sync_copy(zs_v, acc_sh.at[pl.ds(rbase + k * _ZR, _ZR)])
    pltpu.sync_copy(dstage_v, deg_sh.at[pl.ds(rbase, _RPT)])
    plsc.subcore_barrier()

    def run_phase(idx_r, idx_base, dst_r_, dst_base, gc, ngroups, do_deg):
      ge = gc * _CH
      def gslice(j):
        return gbuf_v.at[pl.ds(pl.multiple_of(j * _CH, 8), _CH)]
      def sslice(j):
        return sbuf_v.at[pl.ds(pl.multiple_of(j * _CH, 8), _CH)]
      def start(slot, j):
        pltpu.async_copy(x_r.at[gslice(j)], rows[slot], gsem[slot])
      def wait_scatter(slot, j):
        pltpu.make_async_copy(rows[slot], acc_sh.at[sslice(j)],
                              ssem[slot]).wait()
      def wait_deg(j):
        pltpu.make_async_copy(ones_v, deg_sh.at[sslice(j)], dsem).wait()
      def step(slot, slot2, j):
        t = j + 2
        @pl.when(t < gc)
        def _():
          @pl.when(t >= _NS)
          def _():
            wait_scatter(slot2, j - 1)
          start(slot2, t)
        pltpu.make_async_copy(x_r.at[gslice(j)], rows[slot],
                              gsem[slot]).wait()
        pltpu.async_copy(rows[slot], acc_sh.at[sslice(j)], ssem[slot],
                         add=True)
        if do_deg:
          pltpu.async_copy(ones_v, deg_sh.at[sslice(j)], dsem, add=True)
      def group(g, carry):
        o1 = pl.multiple_of(idx_base + g * ge, 8)
        o2 = pl.multiple_of(dst_base + g * ge, 8)
        pltpu.sync_copy(idx_r.at[pl.ds(o1, ge)], gbuf_v.at[pl.ds(0, ge)])
        pltpu.sync_copy(dst_r_.at[pl.ds(o2, ge)], sbuf_v.at[pl.ds(0, ge)])
        start(0, 0)
        start(1, 1)
        def triple(k, carry2):
          j0 = 3 * k
          step(0, 2, j0)
          step(1, 0, j0 + 1)
          step(2, 1, j0 + 2)
          return carry2
        lax.fori_loop(0, gc // _NS, triple, 0)
        for slot, j in ((0, gc - 3), (1, gc - 2), (2, gc - 1)):
          wait_scatter(slot, j)
        if do_deg:
          def dw(i, carry3):
            wait_deg(0)
            return carry3
          lax.fori_loop(0, gc, dw, 0)
        return carry
      lax.fori_loop(0, ngroups, group, 0)

    run_phase(src_r, c * _EPC + s * _EPT, dst_r, c * _EPC + s * _EPT,
              _GC1, _CPT // _GC1, True)
    plsc.subcore_barrier()

    obase = c * _NPAD + rbase
    for k in range(_RPT // _ZR):
      pltpu.sync_copy(acc_sh.at[pl.ds(rbase + k * _ZR, _ZR)], zs_v)
      pltpu.sync_copy(zs_v, agg_o.at[pl.ds(obase + k * _ZR, _ZR)])
    pltpu.sync_copy(deg_sh.at[pl.ds(rbase, _RPT)], dstage_v)
    pltpu.sync_copy(dstage_v, deg_o.at[pl.ds(obase, _RPT)])
    lax.fori_loop(0, _ZR, fill_zs, 0)
    for k in range(_RPT // _ZR):
      pltpu.sync_copy(zs_v, acc_sh.at[pl.ds(rbase + k * _ZR, _ZR)])
    plsc.subcore_barrier()

    run_phase(nbp_r, c * _FPC + s * _FPT, nbdst_r, c * _FPC + s * _FPT,
              _GC2, _CPT2 // _GC2, False)
    plsc.subcore_barrier()

    for k in range(_RPT // _ZR):
      pltpu.sync_copy(acc_sh.at[pl.ds(rbase + k * _ZR, _ZR)], zs_v)
      pltpu.sync_copy(zs_v, nb_o.at[pl.ds(obase + k * _ZR, _ZR)])

  return body(x, srcp, dstp, nbp, nbdst)


def kernel(x, edge_index, neighbors, W1, W2):
  src = edge_index[0]
  dst = edge_index[1]

  npe = _NC * _EPC - _E
  padg = jnp.arange(npe, dtype=jnp.int32) % _N
  pads = _N + (jnp.arange(npe, dtype=jnp.int32) % (_NPAD - _N))
  srcp = jnp.concatenate([src, padg])
  dstp = jnp.concatenate([dst, pads])
  npf = _NC * _FPC - _NF
  padgf = jnp.arange(npf, dtype=jnp.int32) % _N
  padsf = _N + (jnp.arange(npf, dtype=jnp.int32) % (_NPAD - _N))
  nbp = jnp.concatenate([neighbors.reshape(-1), padgf])
  nbd = jnp.repeat(jnp.arange(_N, dtype=jnp.int32), _NB)
  nbdst = jnp.concatenate([nbd, padsf])

  agg, nbsum, deg = _sc_aggregate(x, srcp, dstp, nbp, nbdst)
  z = _mm2(agg, nbsum, deg.reshape(_NC * _NPAD, 1), W1, W2)
  return z[:_N]


# speedup vs baseline: 10.4835x; 1.0529x over previous
import functools

import jax
import jax.numpy as jnp
from jax import lax
from jax.experimental import pallas as pl
from jax.experimental.pallas import tpu as pltpu
from jax.experimental.pallas import tpu_sc as plsc

_N = 10000
_E = 320000
_D = 128
_H = 256
_OUT = 512
_NB = 25
_NPAD = 10240
_NT = 16
_NC = 2
_CH = 80
_CPT = 126
_EPT = _CPT * _CH
_EPC = _EPT * _NT
_GC1 = 42
_CPT2 = 99
_FPT = _CPT2 * _CH
_FPC = _FPT * _NT
_GC2 = 33
_NF = _N * _NB
_GE = _GC1 * _CH
_NS = 3
_RPT = _NPAD // _NT
_ZR = 32


def _mm2(agg, nbsum, deg, w1, w2):
  def body(a0_ref, a1_ref, n0_ref, n1_ref, d0_ref, d1_ref,
           w1_ref, w2_ref, o_ref):
    r = 1.0 / jnp.maximum(d0_ref[...] + d1_ref[...], 1.0)
    u = (a0_ref[...] + a1_ref[...]) * r \
        + (n0_ref[...] + n1_ref[...]) * (1.0 / _NB)
    h2 = jnp.maximum(
        jnp.dot(u, w1_ref[...], preferred_element_type=jnp.float32), 0.0)
    o_ref[...] = jnp.dot(h2, w2_ref[...], preferred_element_type=jnp.float32)
  nblk = _NPAD // 512
  return pl.pallas_call(
      body,
      grid=(nblk,),
      in_specs=[
          pl.BlockSpec((512, _D), lambda i: (i, 0)),
          pl.BlockSpec((512, _D), lambda i: (i + nblk, 0)),
          pl.BlockSpec((512, _D), lambda i: (i, 0)),
          pl.BlockSpec((512, _D), lambda i: (i + nblk, 0)),
          pl.BlockSpec((512, 1), lambda i: (i, 0)),
          pl.BlockSpec((512, 1), lambda i: (i + nblk, 0)),
          pl.BlockSpec((_D, _H), lambda i: (0, 0)),
          pl.BlockSpec((_H, _OUT), lambda i: (0, 0)),
      ],
      out_specs=pl.BlockSpec((512, _OUT), lambda i: (i, 0)),
      out_shape=jax.ShapeDtypeStruct((_N, _OUT), jnp.float32),
  )(agg, agg, nbsum, nbsum, deg, deg, w1, w2)


def _sc_aggregate(x, srcp, dstp, nbp, nbdst):
  mesh = plsc.VectorSubcoreMesh(core_axis_name="c", subcore_axis_name="s")
  out_type = (
      jax.ShapeDtypeStruct((_NC * _NPAD, _D), jnp.float32),
      jax.ShapeDtypeStruct((_NC * _NPAD, _D), jnp.float32),
      jax.ShapeDtypeStruct((_NC * _NPAD,), jnp.float32),
  )
  scratch = [
      pltpu.VMEM_SHARED((_NPAD, _D), jnp.float32),
      pltpu.VMEM_SHARED((_NPAD,), jnp.float32),
      pltpu.VMEM((_GE,), jnp.int32),
      pltpu.VMEM((_GE,), jnp.int32),
      pltpu.VMEM((_CH, _D), jnp.float32),
      pltpu.VMEM((_CH, _D), jnp.float32),
      pltpu.VMEM((_CH, _D), jnp.float32),
      pltpu.VMEM((_ZR, _D), jnp.float32),
      pltpu.VMEM((_RPT,), jnp.float32),
      pltpu.VMEM((_CH,), jnp.float32),
      pltpu.SemaphoreType.DMA,
      pltpu.SemaphoreType.DMA,
      pltpu.SemaphoreType.DMA,
      pltpu.SemaphoreType.DMA,
      pltpu.SemaphoreType.DMA,
      pltpu.SemaphoreType.DMA,
      pltpu.SemaphoreType.DMA,
  ]

  @functools.partial(pl.kernel, out_type=out_type, mesh=mesh,
                     scratch_types=scratch)
  def body(x_r, src_r, dst_r, nbp_r, nbdst_r, agg_o, nb_o, deg_o,
           acc_sh, deg_sh, gbuf_v, sbuf_v,
           rows0_v, rows1_v, rows2_v, zs_v, dstage_v, ones_v,
           g0, g1, g2, s0, s1, s2, dsem):
    rows = (rows0_v, rows1_v, rows2_v)
    gsem = (g0, g1, g2)
    ssem = (s0, s1, s2)
    c = lax.axis_index("c")
    s = lax.axis_index("s")
    rbase = s * _RPT
    zvec = jnp.zeros((16,), jnp.float32)

    def fill_zs(i, carry):
      for j in range(_D // 16):
        zs_v[i, pl.ds(j * 16, 16)] = zvec
      return carry

    lax.fori_loop(0, _ZR, fill_zs, 0)
    def zdrow(i, carry):
      dstage_v[pl.ds(i * 16, 16)] = zvec
      return carry
    lax.fori_loop(0, _RPT // 16, zdrow, 0)
    ovec = jnp.ones((16,), jnp.float32)
    def orow(i, carry):
      ones_v[pl.ds(i * 16, 16)] = ovec
      return carry
    lax.fori_loop(0, _CH // 16, orow, 0)

    for k in range(_RPT // _ZR):
      pltpu.sync_copy(zs_v, acc_sh.at[pl.ds(rbase + k * _ZR, _ZR)])
    pltpu.sync_copy(dstage_v, deg_sh.at[pl.ds(rbase, _RPT)])
    plsc.subcore_barrier()

    def run_phase(idx_r, idx_base, dst_r_, dst_base, gc, ngroups, do_deg):
      ge = gc * _CH
      def gslice(j):
        return gbuf_v.at[pl.ds(pl.multiple_of(j * _CH, 8), _CH)]
      def sslice(j):
        return sbuf_v.at[pl.ds(pl.multiple_of(j * _CH, 8), _CH)]
      def start(slot, j):
        pltpu.async_copy(x_r.at[gslice(j)], rows[slot], gsem[slot])
      def wait_scatter(slot, j):
        pltpu.make_async_copy(rows[slot], acc_sh.at[sslice(j)],
                              ssem[slot]).wait()
      def wait_deg(j):
        pltpu.make_async_copy(ones_v, deg_sh.at[sslice(j)], dsem).wait()
      def step(slot, slot2, j):
        t = j + 2
        @pl.when(t < gc)
        def _():
          @pl.when(t >= _NS)
          def _():
            wait_scatter(slot2, j - 1)
          start(slot2, t)
        pltpu.make_async_copy(x_r.at[gslice(j)], rows[slot],
                              gsem[slot]).wait()
        pltpu.async_copy(rows[slot], acc_sh.at[sslice(j)], ssem[slot],
                         add=True)
        if do_deg:
          pltpu.async_copy(ones_v, deg_sh.at[sslice(j)], dsem, add=True)
      def group(g, carry):
        o1 = pl.multiple_of(idx_base + g * ge, 8)
        o2 = pl.multiple_of(dst_base + g * ge, 8)
        pltpu.sync_copy(idx_r.at[pl.ds(o1, ge)], gbuf_v.at[pl.ds(0, ge)])
        pltpu.sync_copy(dst_r_.at[pl.ds(o2, ge)], sbuf_v.at[pl.ds(0, ge)])
        start(0, 0)
        start(1, 1)
        def triple(k, carry2):
          j0 = 3 * k
          step(0, 2, j0)
          step(1, 0, j0 + 1)
          step(2, 1, j0 + 2)
          return carry2
        lax.fori_loop(0, gc // _NS, triple, 0)
        for slot, j in ((0, gc - 3), (1, gc - 2), (2, gc - 1)):
          wait_scatter(slot, j)
        if do_deg:
          def dw(i, carry3):
            wait_deg(0)
            return carry3
          lax.fori_loop(0, gc, dw, 0)
        return carry
      lax.fori_loop(0, ngroups, group, 0)

    run_phase(src_r, c * _EPC + s * _EPT, dst_r, c * _EPC + s * _EPT,
              _GC1, _CPT // _GC1, True)
    plsc.subcore_barrier()

    obase = c * _NPAD + rbase
    for k in range(_RPT // _ZR):
      pltpu.sync_copy(acc_sh.at[pl.ds(rbase + k * _ZR, _ZR)], zs_v)
      pltpu.sync_copy(zs_v, agg_o.at[pl.ds(obase + k * _ZR, _ZR)])
    pltpu.sync_copy(deg_sh.at[pl.ds(rbase, _RPT)], dstage_v)
    pltpu.sync_copy(dstage_v, deg_o.at[pl.ds(obase, _RPT)])
    lax.fori_loop(0, _ZR, fill_zs, 0)
    for k in range(_RPT // _ZR):
      pltpu.sync_copy(zs_v, acc_sh.at[pl.ds(rbase + k * _ZR, _ZR)])
    plsc.subcore_barrier()

    run_phase(nbp_r, c * _FPC + s * _FPT, nbdst_r, c * _FPC + s * _FPT,
              _GC2, _CPT2 // _GC2, False)
    plsc.subcore_barrier()

    for k in range(_RPT // _ZR):
      pltpu.sync_copy(acc_sh.at[pl.ds(rbase + k * _ZR, _ZR)], zs_v)
      pltpu.sync_copy(zs_v, nb_o.at[pl.ds(obase + k * _ZR, _ZR)])

  return body(x, srcp, dstp, nbp, nbdst)


def kernel(x, edge_index, neighbors, W1, W2):
  src = edge_index[0]
  dst = edge_index[1]

  npe = _NC * _EPC - _E
  padg = jnp.arange(npe, dtype=jnp.int32) % _N
  pads = _N + (jnp.arange(npe, dtype=jnp.int32) % (_NPAD - _N))
  srcp = jnp.concatenate([src, padg])
  dstp = jnp.concatenate([dst, pads])
  npf = _NC * _FPC - _NF
  padgf = jnp.arange(npf, dtype=jnp.int32) % _N
  padsf = _N + (jnp.arange(npf, dtype=jnp.int32) % (_NPAD - _N))
  nbp = jnp.concatenate([neighbors.reshape(-1), padgf])
  nbd = jnp.repeat(jnp.arange(_N, dtype=jnp.int32), _NB)
  nbdst = jnp.concatenate([nbd, padsf])

  agg, nbsum, deg = _sc_aggregate(x, srcp, dstp, nbp, nbdst)
  return _mm2(agg, nbsum, deg.reshape(_NC * _NPAD, 1), W1, W2)


# speedup vs baseline: 11.5616x; 1.1028x over previous
import functools

import jax
import jax.numpy as jnp
from jax import lax
from jax.experimental import pallas as pl
from jax.experimental.pallas import tpu as pltpu
from jax.experimental.pallas import tpu_sc as plsc

_N = 10000
_E = 320000
_D = 128
_H = 256
_OUT = 512
_NB = 25
_NPAD = 10240
_NT = 16
_NC = 2
_CH = 80
_CPT = 126
_EPT = _CPT * _CH
_EPC = _EPT * _NT
_GC1 = 42
_CPT2 = 99
_FPT = _CPT2 * _CH
_FPC = _FPT * _NT
_GC2 = 33
_NF = _N * _NB
_GE = _GC1 * _CH
_NS = 3
_RPT = _NPAD // _NT
_ZR = 32


def _mm2(agg, nbsum, deg, w1, w2):
  def body(a0_ref, a1_ref, n0_ref, n1_ref, d0_ref, d1_ref,
           w1_ref, w2_ref, o_ref):
    r = 1.0 / jnp.maximum(d0_ref[...] + d1_ref[...], 1.0)
    u = (a0_ref[...] + a1_ref[...]) * r \
        + (n0_ref[...] + n1_ref[...]) * (1.0 / _NB)
    h2 = jnp.maximum(
        jnp.dot(u, w1_ref[...], preferred_element_type=jnp.float32), 0.0)
    o_ref[...] = jnp.dot(h2, w2_ref[...], preferred_element_type=jnp.float32)
  nblk = _NPAD // 512
  return pl.pallas_call(
      body,
      grid=(nblk,),
      in_specs=[
          pl.BlockSpec((512, _D), lambda i: (i, 0)),
          pl.BlockSpec((512, _D), lambda i: (i + nblk, 0)),
          pl.BlockSpec((512, _D), lambda i: (i, 0)),
          pl.BlockSpec((512, _D), lambda i: (i + nblk, 0)),
          pl.BlockSpec((512, 1), lambda i: (i, 0)),
          pl.BlockSpec((512, 1), lambda i: (i + nblk, 0)),
          pl.BlockSpec((_D, _H), lambda i: (0, 0)),
          pl.BlockSpec((_H, _OUT), lambda i: (0, 0)),
      ],
      out_specs=pl.BlockSpec((512, _OUT), lambda i: (i, 0)),
      out_shape=jax.ShapeDtypeStruct((_N, _OUT), jnp.float32),
  )(agg, agg, nbsum, nbsum, deg, deg, w1, w2)


def _sc_aggregate(x, srcp, dstp, nbp, nbdst):
  mesh = plsc.VectorSubcoreMesh(core_axis_name="c", subcore_axis_name="s")
  out_type = (
      jax.ShapeDtypeStruct((_NC * _NPAD, _D), jnp.float32),
      jax.ShapeDtypeStruct((_NC * _NPAD, _D), jnp.float32),
      jax.ShapeDtypeStruct((_NC * _NPAD,), jnp.float32),
  )
  scratch = [
      pltpu.VMEM_SHARED((_NPAD, _D), jnp.float32),
      pltpu.VMEM_SHARED((_NPAD,), jnp.float32),
      pltpu.VMEM((_GE,), jnp.int32),
      pltpu.VMEM((_GE,), jnp.int32),
      pltpu.VMEM((_CH, _D), jnp.float32),
      pltpu.VMEM((_CH, _D), jnp.float32),
      pltpu.VMEM((_CH, _D), jnp.float32),
      pltpu.VMEM((_ZR, _D), jnp.float32),
      pltpu.VMEM((_RPT,), jnp.float32),
      pltpu.VMEM((_CH,), jnp.float32),
      pltpu.SemaphoreType.DMA,
      pltpu.SemaphoreType.DMA,
      pltpu.SemaphoreType.DMA,
      pltpu.SemaphoreType.DMA,
      pltpu.SemaphoreType.DMA,
      pltpu.SemaphoreType.DMA,
      pltpu.SemaphoreType.DMA,
  ]

  @functools.partial(pl.kernel, out_type=out_type, mesh=mesh,
                     scratch_types=scratch)
  def body(x_r, src_r, dst_r, nbp_r, nbdst_r, agg_o, nb_o, deg_o,
           acc_sh, deg_sh, gbuf_v, sbuf_v,
           rows0_v, rows1_v, rows2_v, zs_v, dstage_v, ones_v,
           g0, g1, g2, s0, s1, s2, dsem):
    rows = (rows0_v, rows1_v, rows2_v)
    gsem = (g0, g1, g2)
    ssem = (s0, s1, s2)
    c = lax.axis_index("c")
    s = lax.axis_index("s")
    rbase = s * _RPT
    zvec = jnp.zeros((16,), jnp.float32)

    def fill_zs(i, carry):
      for j in range(_D // 16):
        zs_v[i, pl.ds(j * 16, 16)] = zvec
      return carry

    lax.fori_loop(0, _ZR, fill_zs, 0)
    def zdrow(i, carry):
      dstage_v[pl.ds(i * 16, 16)] = zvec
      return carry
    lax.fori_loop(0, _RPT // 16, zdrow, 0)
    ovec = jnp.ones((16,), jnp.float32)
    def orow(i, carry):
      ones_v[pl.ds(i * 16, 16)] = ovec
      return carry
    lax.fori_loop(0, _CH // 16, orow, 0)

    for k in range(_RPT // _ZR):
      pltpu.sync_copy(zs_v, acc_sh.at[pl.ds(rbase + k * _ZR, _ZR)])
    pltpu.sync_copy(dstage_v, deg_sh.at[pl.ds(rbase, _RPT)])
    plsc.subcore_barrier()

    def run_phase(idx_r, idx_base, dst_r_, dst_base, gc, ngroups, do_deg):
      ge = gc * _CH
      def gslice(j):
        return gbuf_v.at[pl.ds(pl.multiple_of(j * _CH, 8), _CH)]
      def sslice(j):
        return sbuf_v.at[pl.ds(pl.multiple_of(j * _CH, 8), _CH)]
      def start(slot, j):
        pltpu.async_copy(x_r.at[gslice(j)], rows[slot], gsem[slot])
      def wait_scatter(slot, j):
        pltpu.make_async_copy(rows[slot], acc_sh.at[sslice(j)],
                              ssem[slot]).wait()
      def wait_deg(j):
        pltpu.make_async_copy(ones_v, deg_sh.at[sslice(j)], dsem).wait()
      def step(slot, slot2, j):
        t = j + 2
        @pl.when(t < gc)
        def _():
          @pl.when(t >= _NS)
          def _():
            wait_scatter(slot2, j - 1)
          start(slot2, t)
        pltpu.make_async_copy(x_r.at[gslice(j)], rows[slot],
                              gsem[slot]).wait()
        pltpu.async_copy(rows[slot], acc_sh.at[sslice(j)], ssem[slot],
                         add=True)
        if do_deg:
          pltpu.async_copy(ones_v, deg_sh.at[sslice(j)], dsem, add=True)
      def group(g, carry):
        o1 = pl.multiple_of(idx_base + g * ge, 8)
        o2 = pl.multiple_of(dst_base + g * ge, 8)
        pltpu.sync_copy(idx_r.at[pl.ds(o1, ge)], gbuf_v.at[pl.ds(0, ge)])
        pltpu.sync_copy(dst_r_.at[pl.ds(o2, ge)], sbuf_v.at[pl.ds(0, ge)])
        start(0, 0)
        start(1, 1)
        def triple(k, carry2):
          j0 = 3 * k
          step(0, 2, j0)
          step(1, 0, j0 + 1)
          step(2, 1, j0 + 2)
          return carry2
        lax.fori_loop(0, gc // _NS, triple, 0)
        for slot, j in ((0, gc - 3), (1, gc - 2), (2, gc - 1)):
          wait_scatter(slot, j)
        if do_deg:
          def dw(i, carry3):
            wait_deg(0)
            return carry3
          lax.fori_loop(0, gc, dw, 0)
        return carry
      lax.fori_loop(0, ngroups, group, 0)

    run_phase(src_r, c * _EPC + s * _EPT, dst_r, c * _EPC + s * _EPT,
              _GC1, _CPT // _GC1, True)
    plsc.subcore_barrier()

    obase = c * _NPAD + rbase
    for k in range(_RPT // _ZR):
      pltpu.sync_copy(acc_sh.at[pl.ds(rbase + k * _ZR, _ZR)], zs_v)
      pltpu.sync_copy(zs_v, agg_o.at[pl.ds(obase + k * _ZR, _ZR)])
    pltpu.sync_copy(deg_sh.at[pl.ds(rbase, _RPT)], dstage_v)
    pltpu.sync_copy(dstage_v, deg_o.at[pl.ds(obase, _RPT)])
    lax.fori_loop(0, _ZR, fill_zs, 0)
    for k in range(_RPT // _ZR):
      pltpu.sync_copy(zs_v, acc_sh.at[pl.ds(rbase + k * _ZR, _ZR)])
    plsc.subcore_barrier()

    run_phase(nbp_r, c * _FPC + s * _FPT, nbdst_r, c * _FPC + s * _FPT,
              _GC2, _CPT2 // _GC2, False)
    plsc.subcore_barrier()

    for k in range(_RPT // _ZR):
      pltpu.sync_copy(acc_sh.at[pl.ds(rbase + k * _ZR, _ZR)], zs_v)
      pltpu.sync_copy(zs_v, nb_o.at[pl.ds(obase + k * _ZR, _ZR)])

  return body(x, srcp, dstp, nbp, nbdst)


def kernel(x, edge_index, neighbors, W1, W2):
  src = edge_index[0]
  dst = edge_index[1]

  npe = _NC * _EPC - _E
  padg = jnp.arange(npe, dtype=jnp.int32) % _N
  pads = _N + (jnp.arange(npe, dtype=jnp.int32) % (_NPAD - _N))
  srcp = jnp.concatenate([src, padg])
  dstp = jnp.concatenate([dst, pads])
  npf = _NC * _FPC - _NF
  padgf = jnp.arange(npf, dtype=jnp.int32) % _N
  padsf = _N + (jnp.arange(npf, dtype=jnp.int32) % (_NPAD - _N))
  nbp = jnp.concatenate([neighbors.T.reshape(-1), padgf])
  nbd = jnp.tile(jnp.arange(_N, dtype=jnp.int32), _NB)
  nbdst = jnp.concatenate([nbd, padsf])

  agg, nbsum, deg = _sc_aggregate(x, srcp, dstp, nbp, nbdst)
  return _mm2(agg, nbsum, deg.reshape(_NC * _NPAD, 1), W1, W2)
